# baseline (speedup 1.0000x reference)
import functools

import jax
import jax.numpy as jnp
from jax.experimental import pallas as pl


def _shift_r(y):
    return jnp.pad(y, ((0, 0), (1, 0)))[:, :-1]


def _shift_l(y):
    return jnp.pad(y, ((0, 0), (0, 1)))[:, 1:]


def _rpn_kernel(x_ref, top_ref, bot_ref, wt_ref, wh_ref, bc_ref, bh_ref,
                cls_ref, bbox_ref, *, C, W, NP, RS, k):
    i = pl.program_id(1)
    n_i = pl.num_programs(1)
    top = jnp.where(i > 0, top_ref[0], 0.0)
    bot = jnp.where(i < n_i - 1, bot_ref[0], 0.0)

    xc = jnp.concatenate([top, x_ref[0], bot], axis=1).astype(jnp.bfloat16)
    lane = jax.lax.broadcasted_iota(jnp.int32, (1, NP + 2 * W), 1) % W
    x_zr = jnp.where(lane == W - 1, jnp.bfloat16(0), xc)
    x_zl = jnp.where(lane == 0, jnp.bfloat16(0), xc)
    bufs = (x_zr, xc, x_zl)

    n = RS * W
    f32 = jnp.float32
    for p0 in range(0, NP, n):
        ys = []
        for dx in range(3):
            acc = jnp.zeros((C, n), dtype=f32)
            for dy in range(3):
                a = p0 + dy * W
                acc += jnp.dot(wt_ref[3 * dy + dx], bufs[dx][:, a:a + n],
                               preferred_element_type=f32)
            ys.append(acc)
        t = _shift_r(ys[0]) + ys[1] + _shift_l(ys[2]) + bc_ref[:]
        t = jax.nn.relu(t).astype(jnp.bfloat16)

        u = jnp.dot(wh_ref[:], t, preferred_element_type=f32) + bh_ref[:]
        cls_ref[0, :, p0:p0 + n] = jax.nn.sigmoid(u[0:k])
        bbox_ref[0, :, p0:p0 + n] = u[k:5 * k]


def kernel(features, W_conv, b_conv, W_cls, b_cls, W_bbox, b_bbox):
    B, C, H, W = features.shape
    k = W_cls.shape[0]
    NH = 2
    NP = H * W // NH
    RS = 16

    xf = features.reshape(B, C, H * W)

    wt = jnp.transpose(W_conv, (2, 3, 0, 1)).reshape(9, C, C)
    wt = wt.astype(jnp.bfloat16)
    wh = jnp.concatenate([W_cls[:, :, 0, 0], W_bbox[:, :, 0, 0]], axis=0)
    wh = jnp.pad(wh, ((0, 48 - 5 * k), (0, 0))).astype(jnp.bfloat16)
    bc = b_conv.reshape(C, 1)
    bh = jnp.pad(jnp.concatenate([b_cls, b_bbox]), (0, 48 - 5 * k))
    bh = bh.reshape(48, 1)

    nlb = H
    in_specs = [
        pl.BlockSpec((1, C, NP), lambda b, i: (b, 0, i)),
        pl.BlockSpec((1, C, W),
                     lambda b, i: (b, 0, jnp.maximum(i * (NP // W) - 1, 0))),
        pl.BlockSpec((1, C, W),
                     lambda b, i: (b, 0, jnp.minimum((i + 1) * (NP // W),
                                                     nlb - 1))),
        pl.BlockSpec((9, C, C), lambda b, i: (0, 0, 0)),
        pl.BlockSpec((48, C), lambda b, i: (0, 0)),
        pl.BlockSpec((C, 1), lambda b, i: (0, 0)),
        pl.BlockSpec((48, 1), lambda b, i: (0, 0)),
    ]
    out_specs = [
        pl.BlockSpec((1, k, NP), lambda b, i: (b, 0, i)),
        pl.BlockSpec((1, 4 * k, NP), lambda b, i: (b, 0, i)),
    ]
    out_shape = [
        jax.ShapeDtypeStruct((B, k, H * W), jnp.float32),
        jax.ShapeDtypeStruct((B, 4 * k, H * W), jnp.float32),
    ]
    cls_score, bbox_pred = pl.pallas_call(
        functools.partial(_rpn_kernel, C=C, W=W, NP=NP, RS=RS, k=k),
        grid=(B, NH),
        in_specs=in_specs,
        out_specs=out_specs,
        out_shape=out_shape,
    )(xf, xf, xf, wt, wh, bc, bh)
    return (cls_score.reshape(B, k, H, W), bbox_pred.reshape(B, 4 * k, H, W))


# speedup vs baseline: 1.5052x; 1.5052x over previous
import functools

import jax
import jax.numpy as jnp
from jax.experimental import pallas as pl


def _shift_r(y):
    return jnp.pad(y, ((0, 0), (1, 0)))[:, :-1]


def _shift_l(y):
    return jnp.pad(y, ((0, 0), (0, 1)))[:, 1:]


def _rpn_kernel(x_ref, top_ref, bot_ref, wt_ref, wh_ref, bc_ref, bh_ref,
                cls_ref, bbox_ref, *, C, W, Rb, RS, k):
    i = pl.program_id(1)
    n_i = pl.num_programs(1)
    top = jnp.where(i > 0, top_ref[0, :, 7:8, :], 0.0)
    bot = jnp.where(i < n_i - 1, bot_ref[0, :, 0:1, :], 0.0)

    NP = Rb * W
    xe = jnp.concatenate([top, x_ref[0], bot], axis=1).astype(jnp.bfloat16)
    xc = xe.reshape(C, NP + 2 * W)
    lane = jax.lax.broadcasted_iota(jnp.int32, (1, NP + 2 * W), 1) % W
    x_zr = jnp.where(lane == W - 1, jnp.bfloat16(0), xc)
    x_zl = jnp.where(lane == 0, jnp.bfloat16(0), xc)
    bufs = (x_zr, xc, x_zl)

    n = RS * W
    f32 = jnp.float32
    for p0 in range(0, NP, n):
        ys = []
        for dx in range(3):
            acc = jnp.zeros((C, n), dtype=f32)
            for dy in range(3):
                a = p0 + dy * W
                acc += jnp.dot(wt_ref[3 * dy + dx], bufs[dx][:, a:a + n],
                               preferred_element_type=f32)
            ys.append(acc)
        t = _shift_r(ys[0]) + ys[1] + _shift_l(ys[2]) + bc_ref[:]
        t = jax.nn.relu(t).astype(jnp.bfloat16)

        u = jnp.dot(wh_ref[:], t, preferred_element_type=f32) + bh_ref[:]
        u = u.reshape(48, RS, W)
        r0 = p0 // W
        cls_ref[0, :, r0:r0 + RS, :] = jax.nn.sigmoid(u[0:k])
        bbox_ref[0, :, r0:r0 + RS, :] = u[k:5 * k]


def kernel(features, W_conv, b_conv, W_cls, b_cls, W_bbox, b_bbox):
    B, C, H, W = features.shape
    k = W_cls.shape[0]
    NH = 2
    Rb = H // NH
    RS = 32

    wt = jnp.transpose(W_conv, (2, 3, 0, 1)).reshape(9, C, C)
    wt = wt.astype(jnp.bfloat16)
    wh = jnp.concatenate([W_cls[:, :, 0, 0], W_bbox[:, :, 0, 0]], axis=0)
    wh = jnp.pad(wh, ((0, 48 - 5 * k), (0, 0))).astype(jnp.bfloat16)
    bc = b_conv.reshape(C, 1)
    bh = jnp.pad(jnp.concatenate([b_cls, b_bbox]), (0, 48 - 5 * k))
    bh = bh.reshape(48, 1)

    nh8 = H // 8
    in_specs = [
        pl.BlockSpec((1, C, Rb, W), lambda b, i: (b, 0, i, 0)),
        pl.BlockSpec((1, C, 8, W),
                     lambda b, i: (b, 0, jnp.maximum(i * (Rb // 8) - 1, 0), 0)),
        pl.BlockSpec((1, C, 8, W),
                     lambda b, i: (b, 0,
                                   jnp.minimum((i + 1) * (Rb // 8), nh8 - 1),
                                   0)),
        pl.BlockSpec((9, C, C), lambda b, i: (0, 0, 0)),
        pl.BlockSpec((48, C), lambda b, i: (0, 0)),
        pl.BlockSpec((C, 1), lambda b, i: (0, 0)),
        pl.BlockSpec((48, 1), lambda b, i: (0, 0)),
    ]
    out_specs = [
        pl.BlockSpec((1, k, Rb, W), lambda b, i: (b, 0, i, 0)),
        pl.BlockSpec((1, 4 * k, Rb, W), lambda b, i: (b, 0, i, 0)),
    ]
    out_shape = [
        jax.ShapeDtypeStruct((B, k, H, W), jnp.float32),
        jax.ShapeDtypeStruct((B, 4 * k, H, W), jnp.float32),
    ]
    cls_score, bbox_pred = pl.pallas_call(
        functools.partial(_rpn_kernel, C=C, W=W, Rb=Rb, RS=RS, k=k),
        grid=(B, NH),
        in_specs=in_specs,
        out_specs=out_specs,
        out_shape=out_shape,
    )(features, features, features, wt, wh, bc, bh)
    return (cls_score, bbox_pred)


# speedup vs baseline: 1.7220x; 1.1440x over previous
import functools

import jax
import jax.numpy as jnp
from jax.experimental import pallas as pl


def _rpn_kernel(x_ref, top_ref, bot_ref, wt_ref, wh_ref, bc_ref, bh_ref,
                cls_ref, bbox_ref, *, C, W, Rb, RS, k):
    i = pl.program_id(1)
    n_i = pl.num_programs(1)
    top = jnp.where(i > 0, top_ref[0, :, 7:8, :], 0.0)
    bot = jnp.where(i < n_i - 1, bot_ref[0, :, 0:1, :], 0.0)

    n = RS * W
    nw = n + 2 * W
    f32 = jnp.float32
    bz = jnp.bfloat16(0)
    lane = jax.lax.broadcasted_iota(jnp.int32, (1, nw), 1) % W

    for r0 in range(0, Rb, RS):
        if r0 == 0:
            xw = jnp.concatenate([top, x_ref[0, :, 0:RS + 1, :]], axis=1)
        elif r0 + RS == Rb:
            xw = jnp.concatenate([x_ref[0, :, r0 - 1:Rb, :], bot], axis=1)
        else:
            xw = x_ref[0, :, r0 - 1:r0 + RS + 1, :]
        xc = xw.astype(jnp.bfloat16).reshape(C, nw)
        xr = jnp.pad(xc, ((0, 0), (1, 0)))[:, :-1]
        xr = jnp.where(lane == 0, bz, xr)
        xl = jnp.pad(xc, ((0, 0), (0, 1)))[:, 1:]
        xl = jnp.where(lane == W - 1, bz, xl)
        bufs = (xr, xc, xl)

        acc = jnp.zeros((C, n), dtype=f32)
        for dy in range(3):
            a = dy * W
            for dx in range(3):
                acc += jnp.dot(wt_ref[3 * dy + dx], bufs[dx][:, a:a + n],
                               preferred_element_type=f32)
        t = jax.nn.relu(acc + bc_ref[:]).astype(jnp.bfloat16)

        u = jnp.dot(wh_ref[:], t, preferred_element_type=f32) + bh_ref[:]
        u = u.reshape(48, RS, W)
        cls_ref[0, :, r0:r0 + RS, :] = jax.nn.sigmoid(u[0:k])
        bbox_ref[0, :, r0:r0 + RS, :] = u[k:5 * k]


def kernel(features, W_conv, b_conv, W_cls, b_cls, W_bbox, b_bbox):
    B, C, H, W = features.shape
    k = W_cls.shape[0]
    NH = 2
    Rb = H // NH
    RS = 16

    wt = jnp.transpose(W_conv, (2, 3, 0, 1)).reshape(9, C, C)
    wt = wt.astype(jnp.bfloat16)
    wh = jnp.concatenate([W_cls[:, :, 0, 0], W_bbox[:, :, 0, 0]], axis=0)
    wh = jnp.pad(wh, ((0, 48 - 5 * k), (0, 0))).astype(jnp.bfloat16)
    bc = b_conv.reshape(C, 1)
    bh = jnp.pad(jnp.concatenate([b_cls, b_bbox]), (0, 48 - 5 * k))
    bh = bh.reshape(48, 1)

    nh8 = H // 8
    in_specs = [
        pl.BlockSpec((1, C, Rb, W), lambda b, i: (b, 0, i, 0)),
        pl.BlockSpec((1, C, 8, W),
                     lambda b, i: (b, 0, jnp.maximum(i * (Rb // 8) - 1, 0), 0)),
        pl.BlockSpec((1, C, 8, W),
                     lambda b, i: (b, 0,
                                   jnp.minimum((i + 1) * (Rb // 8), nh8 - 1),
                                   0)),
        pl.BlockSpec((9, C, C), lambda b, i: (0, 0, 0)),
        pl.BlockSpec((48, C), lambda b, i: (0, 0)),
        pl.BlockSpec((C, 1), lambda b, i: (0, 0)),
        pl.BlockSpec((48, 1), lambda b, i: (0, 0)),
    ]
    out_specs = [
        pl.BlockSpec((1, k, Rb, W), lambda b, i: (b, 0, i, 0)),
        pl.BlockSpec((1, 4 * k, Rb, W), lambda b, i: (b, 0, i, 0)),
    ]
    out_shape = [
        jax.ShapeDtypeStruct((B, k, H, W), jnp.float32),
        jax.ShapeDtypeStruct((B, 4 * k, H, W), jnp.float32),
    ]
    cls_score, bbox_pred = pl.pallas_call(
        functools.partial(_rpn_kernel, C=C, W=W, Rb=Rb, RS=RS, k=k),
        grid=(B, NH),
        in_specs=in_specs,
        out_specs=out_specs,
        out_shape=out_shape,
    )(features, features, features, wt, wh, bc, bh)
    return (cls_score, bbox_pred)
